# baseline (speedup 1.0000x reference)
import functools
import jax
import jax.numpy as jnp
from jax import lax
from jax.experimental import pallas as pl
from jax.experimental.pallas import tpu as pltpu
from jax.experimental.pallas import tpu_sc as plsc


def kernel(x, pos_embedding):
    B, S, D = x.shape
    NC, NS = 2, 16
    NW = NC * NS
    ROWS = B * S
    RPW = ROWS // NW
    CHUNK = 16
    NCH = RPW // CHUNK
    LANES = 16
    SLICES = D // LANES

    xf = x.reshape(ROWS, D)
    pos = pos_embedding[:S]

    mesh = plsc.VectorSubcoreMesh(core_axis_name="c", subcore_axis_name="s")

    @functools.partial(
        pl.kernel,
        mesh=mesh,
        out_type=jax.ShapeDtypeStruct((ROWS, D), jnp.float32),
        scratch_types=[
            pltpu.VMEM((CHUNK, D), jnp.float32),
            pltpu.VMEM((CHUNK, D), jnp.float32),
        ],
    )
    def sc_add(x_hbm, pos_hbm, out_hbm, xv, pv):
        w = lax.axis_index("s") * NC + lax.axis_index("c")
        row0 = w * RPW
        s0 = lax.rem(row0, S)

        def chunk_body(i, carry):
            r = row0 + i * CHUNK
            sr = s0 + i * CHUNK
            pltpu.sync_copy(x_hbm.at[pl.ds(r, CHUNK)], xv)
            pltpu.sync_copy(pos_hbm.at[pl.ds(sr, CHUNK)], pv)

            def row_body(j, carry2):
                for k in range(SLICES):
                    sl = pl.ds(k * LANES, LANES)
                    xv[j, sl] = xv[j, sl] + pv[j, sl]
                return carry2

            lax.fori_loop(0, CHUNK, row_body, 0)
            pltpu.sync_copy(xv, out_hbm.at[pl.ds(r, CHUNK)])
            return carry

        lax.fori_loop(0, NCH, chunk_body, 0)

    out = sc_add(xf, pos)
    return out.reshape(B, S, D)


# speedup vs baseline: 1.6695x; 1.6695x over previous
import functools
import jax
import jax.numpy as jnp
from jax import lax
from jax.experimental import pallas as pl
from jax.experimental.pallas import tpu as pltpu
from jax.experimental.pallas import tpu_sc as plsc


def kernel(x, pos_embedding):
    B, S, D = x.shape
    NC, NS = 2, 16
    NW = NC * NS
    SPW = S // NW
    CH = 8
    NCH = SPW // CH
    LANES = 16
    HALF = D // (2 * LANES)

    pos = pos_embedding[:S]

    mesh = plsc.VectorSubcoreMesh(core_axis_name="c", subcore_axis_name="s")

    @functools.partial(
        pl.kernel,
        mesh=mesh,
        out_type=jax.ShapeDtypeStruct((B, S, D), jnp.float32),
        scratch_types=[
            pltpu.VMEM((CH, D), jnp.float32),
            pltpu.VMEM((B, CH, D), jnp.float32),
            pltpu.SemaphoreType.DMA,
        ],
    )
    def sc_add(x_hbm, pos_hbm, out_hbm, pv, xv, sem):
        w = lax.axis_index("s") * NC + lax.axis_index("c")
        s0 = w * SPW

        def chunk_body(i, carry):
            sbase = s0 + i * CH
            cp_p = pltpu.async_copy(pos_hbm.at[pl.ds(sbase, CH)], pv, sem)
            cp_x = [
                pltpu.async_copy(x_hbm.at[b, pl.ds(sbase, CH)], xv.at[b], sem)
                for b in range(B)
            ]
            cp_p.wait()
            for c in cp_x:
                c.wait()

            def row_body(j, carry2):
                for h in range(2):
                    pvals = [
                        pv[j, pl.ds((h * HALF + k) * LANES, LANES)]
                        for k in range(HALF)
                    ]
                    for b in range(B):
                        for k in range(HALF):
                            sl = pl.ds((h * HALF + k) * LANES, LANES)
                            xv[b, j, sl] = xv[b, j, sl] + pvals[k]
                return carry2

            lax.fori_loop(0, CH, row_body, 0)
            for b in range(B):
                pltpu.sync_copy(xv.at[b], out_hbm.at[b, pl.ds(sbase, CH)])
            return carry

        lax.fori_loop(0, NCH, chunk_body, 0)

    return sc_add(x, pos)


# speedup vs baseline: 2.2741x; 1.3622x over previous
import functools
import jax
import jax.numpy as jnp
from jax import lax
from jax.experimental import pallas as pl
from jax.experimental.pallas import tpu as pltpu
from jax.experimental.pallas import tpu_sc as plsc


def kernel(x, pos_embedding):
    B, S, D = x.shape
    NC, NS = 2, 16
    NW = NC * NS
    SPW = S // NW
    CH = 4
    NCH = SPW // CH
    NBUF = 4
    G = NCH // NBUF
    LANES = 16
    HALF = D // (2 * LANES)

    pos = pos_embedding[:S]

    mesh = plsc.VectorSubcoreMesh(core_axis_name="c", subcore_axis_name="s")

    @functools.partial(
        pl.kernel,
        mesh=mesh,
        out_type=jax.ShapeDtypeStruct((B, S, D), jnp.float32),
        scratch_types=[
            pltpu.VMEM((NBUF, CH, D), jnp.float32),
            pltpu.VMEM((NBUF, B, CH, D), jnp.float32),
        ]
        + [pltpu.SemaphoreType.DMA] * (2 * NBUF),
    )
    def sc_add(x_hbm, pos_hbm, out_hbm, pvb, xvb, *sems):
        lds, sts = sems[:NBUF], sems[NBUF:]
        w = lax.axis_index("s") * NC + lax.axis_index("c")
        s_w = w * SPW

        def start_load(c, k):
            sb = s_w + c * CH
            pltpu.async_copy(pos_hbm.at[pl.ds(sb, CH)], pvb.at[k], lds[k])
            for b in range(B):
                pltpu.async_copy(x_hbm.at[b, pl.ds(sb, CH)], xvb.at[k, b], lds[k])

        def wait_load(k):
            pltpu.make_async_copy(pos_hbm.at[pl.ds(s_w, CH)], pvb.at[k], lds[k]).wait()
            for b in range(B):
                pltpu.make_async_copy(
                    x_hbm.at[b, pl.ds(s_w, CH)], xvb.at[k, b], lds[k]
                ).wait()

        def start_store(c, k):
            sb = s_w + c * CH
            for b in range(B):
                pltpu.async_copy(xvb.at[k, b], out_hbm.at[b, pl.ds(sb, CH)], sts[k])

        def wait_store(k):
            for b in range(B):
                pltpu.make_async_copy(
                    xvb.at[k, b], out_hbm.at[b, pl.ds(s_w, CH)], sts[k]
                ).wait()

        def compute(k):
            def row_body(j, carry):
                for h in range(2):
                    pvals = [
                        pvb[k, j, pl.ds((h * HALF + q) * LANES, LANES)]
                        for q in range(HALF)
                    ]
                    for b in range(B):
                        for q in range(HALF):
                            sl = pl.ds((h * HALF + q) * LANES, LANES)
                            xvb[k, b, j, sl] = xvb[k, b, j, sl] + pvals[q]
                return carry

            lax.fori_loop(0, CH, row_body, 0)

        start_load(0, 0)
        start_load(1, 1)

        def outer_body(g, carry):
            for k in range(NBUF):
                c = g * NBUF + k
                wait_load(k)
                compute(k)
                start_store(c, k)
                kp = (k + 2) % NBUF
                cp = c + 2

                @pl.when(cp < NCH)
                def _prefetch():
                    if k >= 2:
                        wait_store(kp)
                        start_load(cp, kp)
                    else:
                        @pl.when(g > 0)
                        def _drain():
                            wait_store(kp)

                        start_load(cp, kp)

            return carry

        lax.fori_loop(0, G, outer_body, 0)
        for k in range(NBUF):
            wait_store(k)

    return sc_add(x, pos)
